# baseline (speedup 1.0000x reference)
import jax
import jax.numpy as jnp
from jax.experimental import pallas as pl
from jax.experimental.pallas import tpu as pltpu

_BLOCK = 8192


def _plda_block(s_ref, x_ref, u_ref, y_ref, z_ref):
    x = x_ref[...]
    norm = jnp.sqrt(jnp.sum(x * x, axis=1, keepdims=True))
    norm = jnp.maximum(norm, 1e-12)
    y = (s_ref[0] / norm) * x
    y_ref[...] = y
    z_ref[...] = jnp.dot(y, u_ref[...], preferred_element_type=jnp.float32)


def kernel(x, norm_scale, Ulda):
    batch, dim = x.shape
    scale = jnp.reshape(norm_scale.astype(jnp.float32), (1,))
    grid = (batch // _BLOCK,)
    y, z = pl.pallas_call(
        _plda_block,
        grid=grid,
        in_specs=[
            pl.BlockSpec(memory_space=pltpu.SMEM),
            pl.BlockSpec((_BLOCK, dim), lambda i: (i, 0)),
            pl.BlockSpec((dim, dim), lambda i: (0, 0)),
        ],
        out_specs=[
            pl.BlockSpec((_BLOCK, dim), lambda i: (i, 0)),
            pl.BlockSpec((_BLOCK, dim), lambda i: (i, 0)),
        ],
        out_shape=[
            jax.ShapeDtypeStruct((batch, dim), jnp.float32),
            jax.ShapeDtypeStruct((batch, dim), jnp.float32),
        ],
        compiler_params=pltpu.CompilerParams(
            dimension_semantics=("arbitrary",),
        ),
    )(scale, x, Ulda)
    return (y, z)


# speedup vs baseline: 1.0104x; 1.0104x over previous
import jax
import jax.numpy as jnp
from jax.experimental import pallas as pl
from jax.experimental.pallas import tpu as pltpu

_BLOCK = 4096


def _plda_block(s_ref, x_ref, u_ref, y_ref, z_ref):
    x = x_ref[...]
    norm = jnp.sqrt(jnp.sum(x * x, axis=1, keepdims=True))
    norm = jnp.maximum(norm, 1e-12)
    y = (s_ref[0] / norm) * x
    y_ref[...] = y
    z_ref[...] = jnp.dot(y, u_ref[...], preferred_element_type=jnp.float32)


def kernel(x, norm_scale, Ulda):
    batch, dim = x.shape
    scale = jnp.reshape(norm_scale.astype(jnp.float32), (1,))
    grid = (batch // _BLOCK,)
    y, z = pl.pallas_call(
        _plda_block,
        grid=grid,
        in_specs=[
            pl.BlockSpec(memory_space=pltpu.SMEM),
            pl.BlockSpec((_BLOCK, dim), lambda i: (i, 0)),
            pl.BlockSpec((dim, dim), lambda i: (0, 0)),
        ],
        out_specs=[
            pl.BlockSpec((_BLOCK, dim), lambda i: (i, 0)),
            pl.BlockSpec((_BLOCK, dim), lambda i: (i, 0)),
        ],
        out_shape=[
            jax.ShapeDtypeStruct((batch, dim), jnp.float32),
            jax.ShapeDtypeStruct((batch, dim), jnp.float32),
        ],
        compiler_params=pltpu.CompilerParams(
            dimension_semantics=("arbitrary",),
        ),
    )(scale, x, Ulda)
    return (y, z)
